# baseline (speedup 1.0000x reference)
import functools
import math

import jax
import jax.numpy as jnp
from jax import lax
from jax.experimental import pallas as pl
from jax.experimental.pallas import tpu as pltpu
from jax.experimental.pallas import tpu_sc as plsc

HIDDEN = 128
NUM_GAUSSIANS = 50
NUM_FILTERS = 128
CUTOFF = 10.0
N_NODES = 10000
N_EDGES = 320000

NC = 2
NS = 16
NW = NC * NS
LANES = 16

CHUNK = 128
CHUNKS_PER_WORKER = 79
EDGES_PER_WORKER = CHUNKS_PER_WORKER * CHUNK
E_PAD = NW * EDGES_PER_WORKER
AGG_CHUNKS = 158
AGG_EDGES_PER_WORKER = AGG_CHUNKS * CHUNK
GK = 64

_GAUSS_DELTA = CUTOFF / (NUM_GAUSSIANS - 1)
_GAUSS_COEFF = -0.5 / _GAUSS_DELTA**2
_LOG2 = math.log(2.0)

_mesh = plsc.VectorSubcoreMesh(core_axis_name="c", subcore_axis_name="s")
_mesh1 = plsc.VectorSubcoreMesh(core_axis_name="c", subcore_axis_name="s", num_cores=1)
_sc_params = pltpu.CompilerParams(needs_layout_passes=False, use_tc_tiling_on_sc=False)


@functools.partial(
    pl.kernel,
    out_type=jax.ShapeDtypeStruct((E_PAD,), jnp.float32),
    mesh=_mesh,
    compiler_params=_sc_params,
    scratch_types=[
        pltpu.VMEM((N_NODES,), jnp.float32),
        pltpu.VMEM((N_NODES,), jnp.float32),
        pltpu.VMEM((N_NODES,), jnp.float32),
        pltpu.VMEM((EDGES_PER_WORKER,), jnp.int32),
        pltpu.VMEM((EDGES_PER_WORKER,), jnp.int32),
        pltpu.VMEM((EDGES_PER_WORKER,), jnp.float32),
    ],
)
def _d2_kernel(posx_hbm, posy_hbm, posz_hbm, row_hbm, col_hbm, out_hbm,
               px, py, pz, rv, cv, dv):
    wid = lax.axis_index("s") * NC + lax.axis_index("c")
    base = wid * EDGES_PER_WORKER
    pltpu.sync_copy(posx_hbm, px)
    pltpu.sync_copy(posy_hbm, py)
    pltpu.sync_copy(posz_hbm, pz)
    pltpu.sync_copy(row_hbm.at[pl.ds(base, EDGES_PER_WORKER)], rv)
    pltpu.sync_copy(col_hbm.at[pl.ds(base, EDGES_PER_WORKER)], cv)

    def vec_body(g, carry):
        s = pl.ds(g * LANES, LANES)
        r16 = rv[s]
        c16 = cv[s]
        dx = plsc.load_gather(px, [r16]) - plsc.load_gather(px, [c16])
        dy = plsc.load_gather(py, [r16]) - plsc.load_gather(py, [c16])
        dz = plsc.load_gather(pz, [r16]) - plsc.load_gather(pz, [c16])
        dv[s] = dx * dx + dy * dy + dz * dz
        return carry

    lax.fori_loop(0, EDGES_PER_WORKER // LANES, vec_body, 0)
    pltpu.sync_copy(dv, out_hbm.at[pl.ds(base, EDGES_PER_WORKER)])


_EROWS = E_PAD // 128
_VROWS = N_EDGES // 128


def _scal_body(d2_ref, ew_ref, cc_ref):
    ew = jnp.sqrt(d2_ref[...])
    cutc = 0.5 * (jnp.cos(ew * (math.pi / CUTOFF)) + 1.0)
    rid = lax.broadcasted_iota(jnp.int32, (_EROWS, 128), 0)
    valid = (rid < _VROWS).astype(jnp.float32)
    ew_ref[...] = ew
    cc_ref[...] = cutc * valid


_scal_kernel = pl.pallas_call(
    _scal_body,
    out_shape=(
        jax.ShapeDtypeStruct((_EROWS, 128), jnp.float32),
        jax.ShapeDtypeStruct((_EROWS, 128), jnp.float32),
    ),
)


_BE = 2048


def _w_body(ew_ref, cc_ref, w1_ref, b1_ref, w2_ref, b2_ref, out_ref):
    ew = ew_ref[...]
    offs = lax.broadcasted_iota(jnp.int32, (1, GK), 1).astype(jnp.float32) * _GAUSS_DELTA
    attr = jnp.exp(_GAUSS_COEFF * (ew - offs) ** 2)
    t = attr @ w1_ref[...] + b1_ref[...]
    h1 = jnp.log(0.5 * (1.0 + jnp.exp(t)))
    w = h1 @ w2_ref[...] + b2_ref[...]
    out_ref[...] = w * cc_ref[...]


_w_kernel = pl.pallas_call(
    _w_body,
    grid=(E_PAD // _BE,),
    in_specs=[
        pl.BlockSpec((_BE, 1), lambda i: (i, 0)),
        pl.BlockSpec((_BE, 1), lambda i: (i, 0)),
        pl.BlockSpec((GK, NUM_FILTERS), lambda i: (0, 0)),
        pl.BlockSpec((1, NUM_FILTERS), lambda i: (0, 0)),
        pl.BlockSpec((NUM_FILTERS, NUM_FILTERS), lambda i: (0, 0)),
        pl.BlockSpec((1, NUM_FILTERS), lambda i: (0, 0)),
    ],
    out_specs=pl.BlockSpec((_BE, NUM_FILTERS), lambda i: (i, 0)),
    out_shape=jax.ShapeDtypeStruct((E_PAD, NUM_FILTERS), jnp.float32),
)


N_ACC = 10112
_ROWS_PER_TILE = N_ACC // NS


@functools.partial(
    pl.kernel,
    out_type=jax.ShapeDtypeStruct((N_ACC, HIDDEN), jnp.float32),
    mesh=_mesh1,
    compiler_params=_sc_params,
    scratch_types=[
        pltpu.VMEM((CHUNK,), jnp.int32),
        pltpu.VMEM((CHUNK,), jnp.int32),
        pltpu.VMEM((CHUNK,), jnp.int32),
        pltpu.VMEM((CHUNK,), jnp.int32),
        pltpu.VMEM((CHUNK, HIDDEN), jnp.float32),
        pltpu.VMEM((CHUNK, HIDDEN), jnp.float32),
        pltpu.VMEM((CHUNK, HIDDEN), jnp.float32),
        pltpu.VMEM_SHARED((N_ACC, HIDDEN), jnp.float32),
        pltpu.SemaphoreType.DMA,
        pltpu.SemaphoreType.DMA,
        pltpu.SemaphoreType.DMA,
        pltpu.SemaphoreType.DMA,
        pltpu.SemaphoreType.DMA,
    ],
)
def _agg_kernel(x_hbm, w_hbm, row_hbm, col_hbm, out_hbm,
                rv0, rv1, cv0, cv1, xv0, xv1, wv, acc,
                sg0, sg1, ss0, ss1, sw):
    wid = lax.axis_index("s")
    tile_rows = pl.ds(wid * _ROWS_PER_TILE, _ROWS_PER_TILE)

    def zero_body(r, c2):
        for cc in range(HIDDEN // LANES):
            wv[r, pl.ds(cc * LANES, LANES)] = jnp.zeros((LANES,), jnp.float32)
        return c2

    lax.fori_loop(0, CHUNK, zero_body, 0)
    zfull = _ROWS_PER_TILE // CHUNK
    for k in range(zfull):
        pltpu.sync_copy(
            wv, acc.at[pl.ds(wid * _ROWS_PER_TILE + k * CHUNK, CHUNK)])
    zrem = _ROWS_PER_TILE - zfull * CHUNK
    if zrem:
        pltpu.sync_copy(
            wv.at[pl.ds(0, zrem)],
            acc.at[pl.ds(wid * _ROWS_PER_TILE + zfull * CHUNK, zrem)])
    plsc.subcore_barrier()

    base = wid * AGG_EDGES_PER_WORKER
    rbufs = (rv0, rv1)
    cbufs = (cv0, cv1)
    xbufs = (xv0, xv1)
    gsems = (sg0, sg1)
    ssems = (ss0, ss1)

    def fetch_and_fire(ci, b):
        pltpu.sync_copy(row_hbm.at[pl.ds(base + ci * CHUNK, CHUNK)], rbufs[b])
        pltpu.sync_copy(col_hbm.at[pl.ds(base + ci * CHUNK, CHUNK)], cbufs[b])
        pltpu.async_copy(x_hbm.at[rbufs[b]], xbufs[b], gsems[b])

    def wait_scatter(b):
        pltpu.make_async_copy(xbufs[b], acc.at[cbufs[b]], ssems[b]).wait()

    fetch_and_fire(0, 0)
    fetch_and_fire(1, 1)

    def chunk_body(i, carry):
        for b in range(2):
            ci = i * 2 + b
            xv = xbufs[b]
            wa = pltpu.async_copy(w_hbm.at[pl.ds(base + ci * CHUNK, CHUNK)],
                                  wv, sw)
            pltpu.make_async_copy(x_hbm.at[rbufs[b]], xv, gsems[b]).wait()
            wa.wait()

            def mul_body(r, c2):
                for cc in range(HIDDEN // LANES):
                    s = pl.ds(cc * LANES, LANES)
                    xv[r, s] = xv[r, s] * wv[r, s]
                return c2

            lax.fori_loop(0, CHUNK, mul_body, 0, unroll=2)
            pltpu.async_copy(xv, acc.at[cbufs[b]], ssems[b], add=True)

            @pl.when(ci + 2 < AGG_CHUNKS)
            def _():
                wait_scatter(b)
                fetch_and_fire(ci + 2, b)
        return carry

    lax.fori_loop(0, AGG_CHUNKS // 2, chunk_body, 0)
    wait_scatter(0)
    wait_scatter(1)
    plsc.subcore_barrier()
    pltpu.sync_copy(acc.at[tile_rows], out_hbm.at[tile_rows])


_BN = 2000


def _pre_body(h_ref, w_ref, o_ref):
    o_ref[...] = h_ref[...] @ w_ref[...]


_pre_kernel = pl.pallas_call(
    _pre_body,
    grid=(N_NODES // _BN,),
    in_specs=[
        pl.BlockSpec((_BN, HIDDEN), lambda i: (i, 0)),
        pl.BlockSpec((HIDDEN, NUM_FILTERS), lambda i: (0, 0)),
    ],
    out_specs=pl.BlockSpec((_BN, NUM_FILTERS), lambda i: (i, 0)),
    out_shape=jax.ShapeDtypeStruct((N_NODES, NUM_FILTERS), jnp.float32),
)


def _post_body(a_ref, w2_ref, b2_ref, lw_ref, lb_ref, o_ref):
    t = a_ref[...] @ w2_ref[...] + b2_ref[...]
    t = jax.nn.softplus(t) - _LOG2
    o_ref[...] = t @ lw_ref[...] + lb_ref[...]


_post_kernel = pl.pallas_call(
    _post_body,
    grid=(N_NODES // _BN,),
    in_specs=[
        pl.BlockSpec((_BN, NUM_FILTERS), lambda i: (i, 0)),
        pl.BlockSpec((NUM_FILTERS, HIDDEN), lambda i: (0, 0)),
        pl.BlockSpec((1, HIDDEN), lambda i: (0, 0)),
        pl.BlockSpec((HIDDEN, HIDDEN), lambda i: (0, 0)),
        pl.BlockSpec((1, HIDDEN), lambda i: (0, 0)),
    ],
    out_specs=pl.BlockSpec((_BN, HIDDEN), lambda i: (i, 0)),
    out_shape=jax.ShapeDtypeStruct((N_NODES, HIDDEN), jnp.float32),
)


def kernel(h, pos, edge_index, lin1_w, mlp_w1, mlp_b1, mlp_w2, mlp_b2,
           lin2_w, lin2_b, lin_w, lin_b):
    row = edge_index[0].astype(jnp.int32)
    col = edge_index[1].astype(jnp.int32)
    pad = E_PAD - N_EDGES
    row = jnp.concatenate([row, jnp.zeros((pad,), jnp.int32)])
    col = jnp.concatenate([col, jnp.zeros((pad,), jnp.int32)])

    posx = pos[:, 0]
    posy = pos[:, 1]
    posz = pos[:, 2]

    x = _pre_kernel(h, lin1_w)
    d2 = _d2_kernel(posx, posy, posz, row, col)
    ew2, cc2 = _scal_kernel(d2.reshape(_EROWS, 128))
    w1p = jnp.concatenate(
        [mlp_w1, jnp.zeros((GK - NUM_GAUSSIANS, NUM_FILTERS), jnp.float32)], axis=0)
    w_edge = _w_kernel(ew2.reshape(E_PAD, 1), cc2.reshape(E_PAD, 1),
                       w1p, mlp_b1.reshape(1, -1), mlp_w2, mlp_b2.reshape(1, -1))
    parts = _agg_kernel(x, w_edge, row, col)
    h_update = _post_kernel(parts, lin2_w, lin2_b.reshape(1, -1),
                            lin_w, lin_b.reshape(1, -1))
    return (h_update, pos)


# speedup vs baseline: 1.5106x; 1.5106x over previous
import functools
import math

import jax
import jax.numpy as jnp
from jax import lax
from jax.experimental import pallas as pl
from jax.experimental.pallas import tpu as pltpu
from jax.experimental.pallas import tpu_sc as plsc

HIDDEN = 128
NUM_GAUSSIANS = 50
NUM_FILTERS = 128
CUTOFF = 10.0
N_NODES = 10000
N_EDGES = 320000

NC = 2
NS = 16
NW = NC * NS
LANES = 16

CHUNK = 128
CHUNKS_PER_WORKER = 79
EDGES_PER_WORKER = CHUNKS_PER_WORKER * CHUNK
E_PAD = NW * EDGES_PER_WORKER
AGG_CHUNKS = 158
AGG_EDGES_PER_WORKER = AGG_CHUNKS * CHUNK
GK = 64

_GAUSS_DELTA = CUTOFF / (NUM_GAUSSIANS - 1)
_GAUSS_COEFF = -0.5 / _GAUSS_DELTA**2
_LOG2 = math.log(2.0)

_mesh = plsc.VectorSubcoreMesh(core_axis_name="c", subcore_axis_name="s")
_mesh1 = plsc.VectorSubcoreMesh(core_axis_name="c", subcore_axis_name="s", num_cores=1)
_sc_params = pltpu.CompilerParams(needs_layout_passes=False, use_tc_tiling_on_sc=False)


@functools.partial(
    pl.kernel,
    out_type=jax.ShapeDtypeStruct((E_PAD,), jnp.float32),
    mesh=_mesh,
    compiler_params=_sc_params,
    scratch_types=[
        pltpu.VMEM((N_NODES,), jnp.float32),
        pltpu.VMEM((N_NODES,), jnp.float32),
        pltpu.VMEM((N_NODES,), jnp.float32),
        pltpu.VMEM((EDGES_PER_WORKER,), jnp.int32),
        pltpu.VMEM((EDGES_PER_WORKER,), jnp.int32),
        pltpu.VMEM((EDGES_PER_WORKER,), jnp.float32),
    ],
)
def _d2_kernel(posx_hbm, posy_hbm, posz_hbm, row_hbm, col_hbm, out_hbm,
               px, py, pz, rv, cv, dv):
    wid = lax.axis_index("s") * NC + lax.axis_index("c")
    base = wid * EDGES_PER_WORKER
    pltpu.sync_copy(posx_hbm, px)
    pltpu.sync_copy(posy_hbm, py)
    pltpu.sync_copy(posz_hbm, pz)
    pltpu.sync_copy(row_hbm.at[pl.ds(base, EDGES_PER_WORKER)], rv)
    pltpu.sync_copy(col_hbm.at[pl.ds(base, EDGES_PER_WORKER)], cv)

    def vec_body(g, carry):
        s = pl.ds(g * LANES, LANES)
        r16 = rv[s]
        c16 = cv[s]
        dx = plsc.load_gather(px, [r16]) - plsc.load_gather(px, [c16])
        dy = plsc.load_gather(py, [r16]) - plsc.load_gather(py, [c16])
        dz = plsc.load_gather(pz, [r16]) - plsc.load_gather(pz, [c16])
        dv[s] = dx * dx + dy * dy + dz * dz
        return carry

    lax.fori_loop(0, EDGES_PER_WORKER // LANES, vec_body, 0)
    pltpu.sync_copy(dv, out_hbm.at[pl.ds(base, EDGES_PER_WORKER)])


_EROWS = E_PAD // 128
_VROWS = N_EDGES // 128


def _scal_body(d2_ref, ew_ref, cc_ref):
    ew = jnp.sqrt(d2_ref[...])
    cutc = 0.5 * (jnp.cos(ew * (math.pi / CUTOFF)) + 1.0)
    rid = lax.broadcasted_iota(jnp.int32, (_EROWS, 128), 0)
    valid = (rid < _VROWS).astype(jnp.float32)
    ew_ref[...] = ew
    cc_ref[...] = cutc * valid


_scal_kernel = pl.pallas_call(
    _scal_body,
    out_shape=(
        jax.ShapeDtypeStruct((_EROWS, 128), jnp.float32),
        jax.ShapeDtypeStruct((_EROWS, 128), jnp.float32),
    ),
)


_BE = 2048


def _w_body(ew_ref, cc_ref, w1_ref, b1_ref, w2_ref, b2_ref, out_ref):
    ew = ew_ref[...]
    offs = lax.broadcasted_iota(jnp.int32, (1, GK), 1).astype(jnp.float32) * _GAUSS_DELTA
    attr = jnp.exp(_GAUSS_COEFF * (ew - offs) ** 2)
    t = attr @ w1_ref[...] + b1_ref[...]
    h1 = jnp.log(0.5 * (1.0 + jnp.exp(t)))
    w = h1 @ w2_ref[...] + b2_ref[...]
    out_ref[...] = w * cc_ref[...]


_w_kernel = pl.pallas_call(
    _w_body,
    grid=(E_PAD // _BE,),
    in_specs=[
        pl.BlockSpec((_BE, 1), lambda i: (i, 0)),
        pl.BlockSpec((_BE, 1), lambda i: (i, 0)),
        pl.BlockSpec((GK, NUM_FILTERS), lambda i: (0, 0)),
        pl.BlockSpec((1, NUM_FILTERS), lambda i: (0, 0)),
        pl.BlockSpec((NUM_FILTERS, NUM_FILTERS), lambda i: (0, 0)),
        pl.BlockSpec((1, NUM_FILTERS), lambda i: (0, 0)),
    ],
    out_specs=pl.BlockSpec((_BE, NUM_FILTERS), lambda i: (i, 0)),
    out_shape=jax.ShapeDtypeStruct((E_PAD, NUM_FILTERS), jnp.float32),
)


N_ACC = 10112
_ROWS_PER_TILE = N_ACC // NS


@functools.partial(
    pl.kernel,
    out_type=jax.ShapeDtypeStruct((N_ACC, HIDDEN), jnp.float32),
    mesh=_mesh1,
    compiler_params=_sc_params,
    scratch_types=[
        pltpu.VMEM((CHUNK,), jnp.int32),
        pltpu.VMEM((CHUNK,), jnp.int32),
        pltpu.VMEM((CHUNK,), jnp.int32),
        pltpu.VMEM((CHUNK,), jnp.int32),
        pltpu.VMEM((CHUNK, HIDDEN), jnp.float32),
        pltpu.VMEM((CHUNK, HIDDEN), jnp.float32),
        pltpu.VMEM((CHUNK, HIDDEN), jnp.float32),
        pltpu.VMEM_SHARED((N_ACC, HIDDEN), jnp.float32),
        pltpu.SemaphoreType.DMA,
        pltpu.SemaphoreType.DMA,
        pltpu.SemaphoreType.DMA,
        pltpu.SemaphoreType.DMA,
        pltpu.SemaphoreType.DMA,
    ],
)
def _agg_kernel(x_hbm, w_hbm, row_hbm, col_hbm, out_hbm,
                rv0, rv1, cv0, cv1, xv0, xv1, wv, acc,
                sg0, sg1, si0, si1, sw):
    wid = lax.axis_index("s")
    tile_rows = pl.ds(wid * _ROWS_PER_TILE, _ROWS_PER_TILE)

    def zero_body(r, c2):
        for cc in range(HIDDEN // LANES):
            wv[r, pl.ds(cc * LANES, LANES)] = jnp.zeros((LANES,), jnp.float32)
        return c2

    lax.fori_loop(0, CHUNK, zero_body, 0)
    zfull = _ROWS_PER_TILE // CHUNK
    for k in range(zfull):
        pltpu.sync_copy(
            wv, acc.at[pl.ds(wid * _ROWS_PER_TILE + k * CHUNK, CHUNK)])
    zrem = _ROWS_PER_TILE - zfull * CHUNK
    if zrem:
        pltpu.sync_copy(
            wv.at[pl.ds(0, zrem)],
            acc.at[pl.ds(wid * _ROWS_PER_TILE + zfull * CHUNK, zrem)])
    plsc.subcore_barrier()

    base = wid * AGG_EDGES_PER_WORKER
    rbufs = (rv0, rv1)
    cbufs = (cv0, cv1)
    xbufs = (xv0, xv1)
    gsems = (sg0, sg1)

    isems = (si0, si1)

    def fetch_and_fire(ci, b):
        ra = pltpu.async_copy(row_hbm.at[pl.ds(base + ci * CHUNK, CHUNK)],
                              rbufs[b], isems[b])
        ca = pltpu.async_copy(col_hbm.at[pl.ds(base + ci * CHUNK, CHUNK)],
                              cbufs[b], isems[b])
        ra.wait()
        ca.wait()
        pltpu.async_copy(x_hbm.at[rbufs[b]], xbufs[b], gsems[b])

    fetch_and_fire(0, 0)
    fetch_and_fire(1, 1)

    def chunk_body(i, carry):
        for b in range(2):
            ci = i * 2 + b
            xv = xbufs[b]
            wa = pltpu.async_copy(w_hbm.at[pl.ds(base + ci * CHUNK, CHUNK)],
                                  wv, sw)
            pltpu.make_async_copy(x_hbm.at[rbufs[b]], xv, gsems[b]).wait()
            wa.wait()

            def mul_body(r, c2):
                for cc in range(HIDDEN // LANES):
                    s = pl.ds(cc * LANES, LANES)
                    xv[r, s] = xv[r, s] * wv[r, s]
                return c2

            lax.fori_loop(0, CHUNK, mul_body, 0)
            pltpu.sync_copy(xv, acc.at[cbufs[b]], add=True)

            @pl.when(ci + 2 < AGG_CHUNKS)
            def _():
                fetch_and_fire(ci + 2, b)
        return carry

    lax.fori_loop(0, AGG_CHUNKS // 2, chunk_body, 0)
    plsc.subcore_barrier()
    pltpu.sync_copy(acc.at[tile_rows], out_hbm.at[tile_rows])


_BN = 2000


def _pre_body(h_ref, w_ref, o_ref):
    o_ref[...] = h_ref[...] @ w_ref[...]


_pre_kernel = pl.pallas_call(
    _pre_body,
    grid=(N_NODES // _BN,),
    in_specs=[
        pl.BlockSpec((_BN, HIDDEN), lambda i: (i, 0)),
        pl.BlockSpec((HIDDEN, NUM_FILTERS), lambda i: (0, 0)),
    ],
    out_specs=pl.BlockSpec((_BN, NUM_FILTERS), lambda i: (i, 0)),
    out_shape=jax.ShapeDtypeStruct((N_NODES, NUM_FILTERS), jnp.float32),
)


def _post_body(a_ref, w2_ref, b2_ref, lw_ref, lb_ref, o_ref):
    t = a_ref[...] @ w2_ref[...] + b2_ref[...]
    t = jax.nn.softplus(t) - _LOG2
    o_ref[...] = t @ lw_ref[...] + lb_ref[...]


_post_kernel = pl.pallas_call(
    _post_body,
    grid=(N_NODES // _BN,),
    in_specs=[
        pl.BlockSpec((_BN, NUM_FILTERS), lambda i: (i, 0)),
        pl.BlockSpec((NUM_FILTERS, HIDDEN), lambda i: (0, 0)),
        pl.BlockSpec((1, HIDDEN), lambda i: (0, 0)),
        pl.BlockSpec((HIDDEN, HIDDEN), lambda i: (0, 0)),
        pl.BlockSpec((1, HIDDEN), lambda i: (0, 0)),
    ],
    out_specs=pl.BlockSpec((_BN, HIDDEN), lambda i: (i, 0)),
    out_shape=jax.ShapeDtypeStruct((N_NODES, HIDDEN), jnp.float32),
)


def kernel(h, pos, edge_index, lin1_w, mlp_w1, mlp_b1, mlp_w2, mlp_b2,
           lin2_w, lin2_b, lin_w, lin_b):
    row = edge_index[0].astype(jnp.int32)
    col = edge_index[1].astype(jnp.int32)
    pad = E_PAD - N_EDGES
    row = jnp.concatenate([row, jnp.zeros((pad,), jnp.int32)])
    col = jnp.concatenate([col, jnp.zeros((pad,), jnp.int32)])

    posx = pos[:, 0]
    posy = pos[:, 1]
    posz = pos[:, 2]

    x = _pre_kernel(h, lin1_w)
    d2 = _d2_kernel(posx, posy, posz, row, col)
    ew2, cc2 = _scal_kernel(d2.reshape(_EROWS, 128))
    w1p = jnp.concatenate(
        [mlp_w1, jnp.zeros((GK - NUM_GAUSSIANS, NUM_FILTERS), jnp.float32)], axis=0)
    w_edge = _w_kernel(ew2.reshape(E_PAD, 1), cc2.reshape(E_PAD, 1),
                       w1p, mlp_b1.reshape(1, -1), mlp_w2, mlp_b2.reshape(1, -1))
    parts = _agg_kernel(x, w_edge, row, col)
    h_update = _post_kernel(parts, lin2_w, lin2_b.reshape(1, -1),
                            lin_w, lin_b.reshape(1, -1))
    return (h_update, pos)


# speedup vs baseline: 1.8534x; 1.2269x over previous
import functools
import math

import jax
import jax.numpy as jnp
from jax import lax
from jax.experimental import pallas as pl
from jax.experimental.pallas import tpu as pltpu
from jax.experimental.pallas import tpu_sc as plsc

HIDDEN = 128
NUM_GAUSSIANS = 50
NUM_FILTERS = 128
CUTOFF = 10.0
N_NODES = 10000
N_EDGES = 320000

NC = 2
NS = 16
NW = NC * NS
LANES = 16

CHUNK = 128
CHUNKS_PER_WORKER = 79
EDGES_PER_WORKER = CHUNKS_PER_WORKER * CHUNK
E_PAD = NW * EDGES_PER_WORKER
AGG_CHUNKS = 158
AGG_EDGES_PER_WORKER = AGG_CHUNKS * CHUNK
GK = 64

_GAUSS_DELTA = CUTOFF / (NUM_GAUSSIANS - 1)
_GAUSS_COEFF = -0.5 / _GAUSS_DELTA**2
_LOG2 = math.log(2.0)

_mesh = plsc.VectorSubcoreMesh(core_axis_name="c", subcore_axis_name="s")
_mesh1 = plsc.VectorSubcoreMesh(core_axis_name="c", subcore_axis_name="s", num_cores=1)
_sc_params = pltpu.CompilerParams(needs_layout_passes=False, use_tc_tiling_on_sc=False)


@functools.partial(
    pl.kernel,
    out_type=jax.ShapeDtypeStruct((E_PAD,), jnp.float32),
    mesh=_mesh,
    compiler_params=_sc_params,
    scratch_types=[
        pltpu.VMEM((N_NODES,), jnp.float32),
        pltpu.VMEM((N_NODES,), jnp.float32),
        pltpu.VMEM((N_NODES,), jnp.float32),
        pltpu.VMEM((EDGES_PER_WORKER,), jnp.int32),
        pltpu.VMEM((EDGES_PER_WORKER,), jnp.int32),
        pltpu.VMEM((EDGES_PER_WORKER,), jnp.float32),
    ],
)
def _d2_kernel(posx_hbm, posy_hbm, posz_hbm, row_hbm, col_hbm, out_hbm,
               px, py, pz, rv, cv, dv):
    wid = lax.axis_index("s") * NC + lax.axis_index("c")
    base = wid * EDGES_PER_WORKER
    pltpu.sync_copy(posx_hbm, px)
    pltpu.sync_copy(posy_hbm, py)
    pltpu.sync_copy(posz_hbm, pz)
    pltpu.sync_copy(row_hbm.at[pl.ds(base, EDGES_PER_WORKER)], rv)
    pltpu.sync_copy(col_hbm.at[pl.ds(base, EDGES_PER_WORKER)], cv)

    def vec_body(g, carry):
        s = pl.ds(g * LANES, LANES)
        r16 = rv[s]
        c16 = cv[s]
        dx = plsc.load_gather(px, [r16]) - plsc.load_gather(px, [c16])
        dy = plsc.load_gather(py, [r16]) - plsc.load_gather(py, [c16])
        dz = plsc.load_gather(pz, [r16]) - plsc.load_gather(pz, [c16])
        dv[s] = dx * dx + dy * dy + dz * dz
        return carry

    lax.fori_loop(0, EDGES_PER_WORKER // LANES, vec_body, 0)
    pltpu.sync_copy(dv, out_hbm.at[pl.ds(base, EDGES_PER_WORKER)])


_EROWS = E_PAD // 128
_VROWS = N_EDGES // 128


def _scal_body(d2_ref, ew_ref, cc_ref):
    ew = jnp.sqrt(d2_ref[...])
    cutc = 0.5 * (jnp.cos(ew * (math.pi / CUTOFF)) + 1.0)
    rid = lax.broadcasted_iota(jnp.int32, (_EROWS, 128), 0)
    valid = (rid < _VROWS).astype(jnp.float32)
    ew_ref[...] = ew
    cc_ref[...] = cutc * valid


_scal_kernel = pl.pallas_call(
    _scal_body,
    out_shape=(
        jax.ShapeDtypeStruct((_EROWS, 128), jnp.float32),
        jax.ShapeDtypeStruct((_EROWS, 128), jnp.float32),
    ),
)


_BE = 2048


def _w_body(ew_ref, cc_ref, w1_ref, b1_ref, w2_ref, b2_ref, out_ref):
    ew = ew_ref[...]
    offs = lax.broadcasted_iota(jnp.int32, (1, GK), 1).astype(jnp.float32) * _GAUSS_DELTA
    attr = jnp.exp(_GAUSS_COEFF * (ew - offs) ** 2)
    t = attr @ w1_ref[...] + b1_ref[...]
    h1 = jnp.log(0.5 * (1.0 + jnp.exp(t)))
    w = h1 @ w2_ref[...] + b2_ref[...]
    out_ref[...] = w * cc_ref[...]


_w_kernel = pl.pallas_call(
    _w_body,
    grid=(E_PAD // _BE,),
    in_specs=[
        pl.BlockSpec((_BE, 1), lambda i: (i, 0)),
        pl.BlockSpec((_BE, 1), lambda i: (i, 0)),
        pl.BlockSpec((GK, NUM_FILTERS), lambda i: (0, 0)),
        pl.BlockSpec((1, NUM_FILTERS), lambda i: (0, 0)),
        pl.BlockSpec((NUM_FILTERS, NUM_FILTERS), lambda i: (0, 0)),
        pl.BlockSpec((1, NUM_FILTERS), lambda i: (0, 0)),
    ],
    out_specs=pl.BlockSpec((_BE, NUM_FILTERS), lambda i: (i, 0)),
    out_shape=jax.ShapeDtypeStruct((E_PAD, NUM_FILTERS), jnp.float32),
)


N_ACC = 10112
_ROWS_PER_TILE = N_ACC // NS


@functools.partial(
    pl.kernel,
    out_type=jax.ShapeDtypeStruct((N_ACC, HIDDEN), jnp.float32),
    mesh=_mesh1,
    compiler_params=_sc_params,
    scratch_types=[
        pltpu.VMEM((CHUNK,), jnp.int32),
        pltpu.VMEM((CHUNK,), jnp.int32),
        pltpu.VMEM((CHUNK,), jnp.int32),
        pltpu.VMEM((CHUNK,), jnp.int32),
        pltpu.VMEM((CHUNK, HIDDEN), jnp.float32),
        pltpu.VMEM((CHUNK, HIDDEN), jnp.float32),
        pltpu.VMEM((CHUNK, HIDDEN), jnp.float32),
        pltpu.VMEM_SHARED((N_ACC, HIDDEN), jnp.float32),
        pltpu.SemaphoreType.DMA,
        pltpu.SemaphoreType.DMA,
        pltpu.SemaphoreType.DMA,
        pltpu.SemaphoreType.DMA,
        pltpu.SemaphoreType.DMA,
    ],
)
def _agg_kernel(x_hbm, w_hbm, row_hbm, col_hbm, out_hbm,
                rv0, rv1, cv0, cv1, xv0, xv1, wv, acc,
                sg0, sg1, si0, si1, sw):
    wid = lax.axis_index("s")
    tile_rows = pl.ds(wid * _ROWS_PER_TILE, _ROWS_PER_TILE)

    def zero_body(r, c2):
        for cc in range(HIDDEN // LANES):
            wv[r, pl.ds(cc * LANES, LANES)] = jnp.zeros((LANES,), jnp.float32)
        return c2

    lax.fori_loop(0, CHUNK, zero_body, 0)
    zfull = _ROWS_PER_TILE // CHUNK
    for k in range(zfull):
        pltpu.sync_copy(
            wv, acc.at[pl.ds(wid * _ROWS_PER_TILE + k * CHUNK, CHUNK)])
    zrem = _ROWS_PER_TILE - zfull * CHUNK
    if zrem:
        pltpu.sync_copy(
            wv.at[pl.ds(0, zrem)],
            acc.at[pl.ds(wid * _ROWS_PER_TILE + zfull * CHUNK, zrem)])
    plsc.subcore_barrier()

    base = wid * AGG_EDGES_PER_WORKER
    rbufs = (rv0, rv1)
    cbufs = (cv0, cv1)
    xbufs = (xv0, xv1)
    gsems = (sg0, sg1)

    isems = (si0, si1)

    def fetch_and_fire(ci, b):
        ra = pltpu.async_copy(row_hbm.at[pl.ds(base + ci * CHUNK, CHUNK)],
                              rbufs[b], isems[b])
        ca = pltpu.async_copy(col_hbm.at[pl.ds(base + ci * CHUNK, CHUNK)],
                              cbufs[b], isems[b])
        ra.wait()
        ca.wait()
        pltpu.async_copy(x_hbm.at[rbufs[b]], xbufs[b], gsems[b])

    def fire_w(ci):
        pltpu.async_copy(w_hbm.at[pl.ds(base + ci * CHUNK, CHUNK)], wv, sw)

    fetch_and_fire(0, 0)
    fetch_and_fire(1, 1)
    fire_w(0)

    def chunk_body(i, carry):
        for b in range(2):
            ci = i * 2 + b
            xv = xbufs[b]
            pltpu.make_async_copy(x_hbm.at[rbufs[b]], xv, gsems[b]).wait()
            pltpu.make_async_copy(w_hbm.at[pl.ds(base + ci * CHUNK, CHUNK)],
                                  wv, sw).wait()

            def mul_body(r, c2):
                for cc in range(HIDDEN // LANES):
                    s = pl.ds(cc * LANES, LANES)
                    xv[r, s] = xv[r, s] * wv[r, s]
                return c2

            lax.fori_loop(0, CHUNK, mul_body, 0)

            @pl.when(ci + 1 < AGG_CHUNKS)
            def _():
                fire_w(ci + 1)

            pltpu.sync_copy(xv, acc.at[cbufs[b]], add=True)

            @pl.when(ci + 2 < AGG_CHUNKS)
            def _():
                fetch_and_fire(ci + 2, b)
        return carry

    lax.fori_loop(0, AGG_CHUNKS // 2, chunk_body, 0)
    plsc.subcore_barrier()
    pltpu.sync_copy(acc.at[tile_rows], out_hbm.at[tile_rows])


_BN = 2000


def _pre_body(h_ref, w_ref, o_ref):
    o_ref[...] = h_ref[...] @ w_ref[...]


_pre_kernel = pl.pallas_call(
    _pre_body,
    grid=(N_NODES // _BN,),
    in_specs=[
        pl.BlockSpec((_BN, HIDDEN), lambda i: (i, 0)),
        pl.BlockSpec((HIDDEN, NUM_FILTERS), lambda i: (0, 0)),
    ],
    out_specs=pl.BlockSpec((_BN, NUM_FILTERS), lambda i: (i, 0)),
    out_shape=jax.ShapeDtypeStruct((N_NODES, NUM_FILTERS), jnp.float32),
)


def _post_body(a_ref, w2_ref, b2_ref, lw_ref, lb_ref, o_ref):
    t = a_ref[...] @ w2_ref[...] + b2_ref[...]
    t = jax.nn.softplus(t) - _LOG2
    o_ref[...] = t @ lw_ref[...] + lb_ref[...]


_post_kernel = pl.pallas_call(
    _post_body,
    grid=(N_NODES // _BN,),
    in_specs=[
        pl.BlockSpec((_BN, NUM_FILTERS), lambda i: (i, 0)),
        pl.BlockSpec((NUM_FILTERS, HIDDEN), lambda i: (0, 0)),
        pl.BlockSpec((1, HIDDEN), lambda i: (0, 0)),
        pl.BlockSpec((HIDDEN, HIDDEN), lambda i: (0, 0)),
        pl.BlockSpec((1, HIDDEN), lambda i: (0, 0)),
    ],
    out_specs=pl.BlockSpec((_BN, HIDDEN), lambda i: (i, 0)),
    out_shape=jax.ShapeDtypeStruct((N_NODES, HIDDEN), jnp.float32),
)


def kernel(h, pos, edge_index, lin1_w, mlp_w1, mlp_b1, mlp_w2, mlp_b2,
           lin2_w, lin2_b, lin_w, lin_b):
    row = edge_index[0].astype(jnp.int32)
    col = edge_index[1].astype(jnp.int32)
    pad = E_PAD - N_EDGES
    row = jnp.concatenate([row, jnp.zeros((pad,), jnp.int32)])
    col = jnp.concatenate([col, jnp.zeros((pad,), jnp.int32)])

    posx = pos[:, 0]
    posy = pos[:, 1]
    posz = pos[:, 2]

    x = _pre_kernel(h, lin1_w)
    d2 = _d2_kernel(posx, posy, posz, row, col)
    ew2, cc2 = _scal_kernel(d2.reshape(_EROWS, 128))
    w1p = jnp.concatenate(
        [mlp_w1, jnp.zeros((GK - NUM_GAUSSIANS, NUM_FILTERS), jnp.float32)], axis=0)
    w_edge = _w_kernel(ew2.reshape(E_PAD, 1), cc2.reshape(E_PAD, 1),
                       w1p, mlp_b1.reshape(1, -1), mlp_w2, mlp_b2.reshape(1, -1))
    parts = _agg_kernel(x, w_edge, row, col)
    h_update = _post_kernel(parts, lin2_w, lin2_b.reshape(1, -1),
                            lin_w, lin_b.reshape(1, -1))
    return (h_update, pos)
